# baseline (speedup 1.0000x reference)
import jax
import jax.numpy as jnp
from jax.experimental import pallas as pl


def _relu_body(x_ref, o_ref):
    o_ref[...] = jnp.maximum(x_ref[...], 0.0)


def kernel(inputs):
    shape = inputs.shape
    n = inputs.size
    rows, cols = 1792, n // 1792
    x2 = inputs.reshape(rows, cols)
    block_rows = 128
    out = pl.pallas_call(
        _relu_body,
        grid=(rows // block_rows,),
        in_specs=[pl.BlockSpec((block_rows, cols), lambda i: (i, 0))],
        out_specs=pl.BlockSpec((block_rows, cols), lambda i: (i, 0)),
        out_shape=jax.ShapeDtypeStruct((rows, cols), jnp.float32),
    )(x2)
    return out.reshape(shape)


# speedup vs baseline: 1.3237x; 1.3237x over previous
import jax
import jax.numpy as jnp
from jax.experimental import pallas as pl


def _relu_body(x_ref, o_ref):
    o_ref[...] = jnp.maximum(x_ref[...], 0.0)


def kernel(inputs):
    shape = inputs.shape
    lead = shape[0] * shape[1] * shape[2]
    x3 = inputs.reshape(lead, shape[3], shape[4])
    block_rows = 112
    out = pl.pallas_call(
        _relu_body,
        grid=(lead // block_rows,),
        in_specs=[pl.BlockSpec((block_rows, shape[3], shape[4]), lambda i: (i, 0, 0))],
        out_specs=pl.BlockSpec((block_rows, shape[3], shape[4]), lambda i: (i, 0, 0)),
        out_shape=jax.ShapeDtypeStruct(x3.shape, jnp.float32),
    )(x3)
    return out.reshape(shape)


# speedup vs baseline: 2.7210x; 2.0556x over previous
import jax
import jax.numpy as jnp
from jax.experimental import pallas as pl


def _relu_body(x_ref, o_ref):
    o_ref[...] = jnp.maximum(x_ref[...], 0.0)


def kernel(inputs):
    shape = inputs.shape
    b0, b1, h, w, c = shape
    block_h = 112
    spec = pl.BlockSpec(
        (1, 1, block_h, w, c), lambda i, j, k: (i, j, k, 0, 0)
    )
    return pl.pallas_call(
        _relu_body,
        grid=(b0, b1, h // block_h),
        in_specs=[spec],
        out_specs=spec,
        out_shape=jax.ShapeDtypeStruct(shape, jnp.float32),
    )(inputs)
